# baseline (speedup 1.0000x reference)
import functools

import jax
import jax.numpy as jnp
from jax import lax
from jax.experimental import pallas as pl
from jax.experimental.pallas import tpu as pltpu
from jax.experimental.pallas import tpu_sc as plsc

N = 16384
C = 1000

_info = plsc.get_sparse_core_info()
_NC, _NS = _info.num_cores, _info.num_subcores
_NW = _NC * _NS
_RPW = N // _NW
_CROWS = 64
_NCHUNK = _RPW // _CROWS
_FULL = C // 16
_TAIL = C - _FULL * 16


def _sc_loss_partials(yh, y):
    mesh = plsc.VectorSubcoreMesh(core_axis_name="c", subcore_axis_name="s")

    @functools.partial(
        pl.kernel,
        mesh=mesh,
        out_type=jax.ShapeDtypeStruct((_NW, 16), jnp.float32),
        scratch_types=[
            pltpu.VMEM((_CROWS, C), jnp.float32),
            pltpu.VMEM((_RPW,), jnp.int32),
            pltpu.VMEM((16,), jnp.float32),
        ],
    )
    def k(yh_hbm, y_hbm, out_hbm, chunk_v, y_v, acc_v):
        wid = lax.axis_index("s") * _NC + lax.axis_index("c")
        row0 = wid * _RPW
        pltpu.sync_copy(y_hbm.at[pl.ds(row0, _RPW)], y_v)

        lane = lax.iota(jnp.int32, 16)
        tail_mask = lane >= (16 - _TAIL)

        acc = jnp.zeros((16,), jnp.float32)
        gacc = jnp.zeros((16,), jnp.float32)
        for ch in range(_NCHUNK):
            pltpu.sync_copy(
                yh_hbm.at[pl.ds(row0 + ch * _CROWS, _CROWS), :], chunk_v
            )

            def row_body(r, a):
                for j in range(_FULL):
                    a = a + chunk_v[r, pl.ds(j * 16, 16)]
                tail = chunk_v[r, pl.ds(C - 16, 16)]
                return a + jnp.where(tail_mask, tail, 0.0)

            acc = lax.fori_loop(0, _CROWS, row_body, acc)

            for g in range(_CROWS // 16):
                y16 = y_v[pl.ds(ch * _CROWS + g * 16, 16)]
                for kk in range(16):
                    yv = y16[kk]
                    base = pl.multiple_of(yv & -16, 16)
                    vec = chunk_v[g * 16 + kk, pl.ds(base, 16)]
                    gacc = gacc + jnp.where(lane == (yv - base), vec, 0.0)

        acc_v[...] = acc - 2.0 * gacc
        pltpu.sync_copy(acc_v, out_hbm.at[wid])

    return k(yh, y)


def kernel(yh, y):
    partials = _sc_loss_partials(yh, y.astype(jnp.int32))
    return partials.sum()


# speedup vs baseline: 1.3956x; 1.3956x over previous
import functools

import jax
import jax.numpy as jnp
from jax import lax
from jax.experimental import pallas as pl
from jax.experimental.pallas import tpu as pltpu
from jax.experimental.pallas import tpu_sc as plsc

N = 16384
C = 1000

_info = plsc.get_sparse_core_info()
_NC, _NS = _info.num_cores, _info.num_subcores
_NW = _NC * _NS
_RPW = N // _NW
_CROWS = 32
_NCHUNK = _RPW // _CROWS
_NPAIR = _NCHUNK // 2
_FULL = C // 16
_TAIL = C - _FULL * 16
_NACC = 8


def _sc_loss_partials(yh, y):
    mesh = plsc.VectorSubcoreMesh(core_axis_name="c", subcore_axis_name="s")

    @functools.partial(
        pl.kernel,
        mesh=mesh,
        out_type=jax.ShapeDtypeStruct((_NW, 16), jnp.float32),
        scratch_types=[
            pltpu.VMEM((_CROWS, C), jnp.float32),
            pltpu.VMEM((_CROWS, C), jnp.float32),
            pltpu.VMEM((_RPW,), jnp.int32),
            pltpu.VMEM((16,), jnp.float32),
            pltpu.SemaphoreType.DMA,
            pltpu.SemaphoreType.DMA,
        ],
    )
    def k(yh_hbm, y_hbm, out_hbm, buf0, buf1, y_v, acc_v, sem0, sem1):
        wid = lax.axis_index("s") * _NC + lax.axis_index("c")
        row0 = wid * _RPW
        pltpu.sync_copy(y_hbm.at[pl.ds(row0, _RPW)], y_v)

        lane = lax.iota(jnp.int32, 16)
        tail_mask = lane >= (16 - _TAIL)

        def start(ch, buf, sem):
            pltpu.async_copy(
                yh_hbm.at[pl.ds(row0 + ch * _CROWS, _CROWS), :], buf, sem
            )

        def drain(buf, sem):
            pltpu.make_async_copy(
                yh_hbm.at[pl.ds(0, _CROWS), :], buf, sem
            ).wait()

        def consume(ch, buf, carry):
            accs, gacc = list(carry[:_NACC]), carry[_NACC]

            def row_body(r, aa):
                aa = list(aa)
                for j in range(_FULL):
                    aa[j % _NACC] = aa[j % _NACC] + buf[r, pl.ds(j * 16, 16)]
                tail = buf[r, pl.ds(C - 16, 16)]
                aa[_NACC - 1] = aa[_NACC - 1] + jnp.where(tail_mask, tail, 0.0)
                return tuple(aa)

            accs = list(lax.fori_loop(0, _CROWS, row_body, tuple(accs)))

            for g in range(_CROWS // 16):
                y16 = y_v[pl.ds(ch * _CROWS + g * 16, 16)]
                for kk in range(16):
                    yv = y16[kk]
                    base = pl.multiple_of(yv & -16, 16)
                    vec = buf[g * 16 + kk, pl.ds(base, 16)]
                    gacc = gacc + jnp.where(lane == (yv - base), vec, 0.0)

            return (*accs, gacc)

        start(0, buf0, sem0)

        def pair_body(p, carry):
            ch0 = p * 2
            start(ch0 + 1, buf1, sem1)
            drain(buf0, sem0)
            carry = consume(ch0, buf0, carry)
            start(jnp.minimum(ch0 + 2, _NCHUNK - 1), buf0, sem0)
            drain(buf1, sem1)
            carry = consume(ch0 + 1, buf1, carry)
            return carry

        zero = jnp.zeros((16,), jnp.float32)
        carry = lax.fori_loop(
            0, _NPAIR, pair_body, tuple([zero] * _NACC + [zero])
        )
        drain(buf0, sem0)

        accs, gacc = carry[:_NACC], carry[_NACC]
        acc = accs[0]
        for a in accs[1:]:
            acc = acc + a
        acc_v[...] = acc - 2.0 * gacc
        pltpu.sync_copy(acc_v, out_hbm.at[wid])

    return k(yh, y)


def kernel(yh, y):
    partials = _sc_loss_partials(yh, y.astype(jnp.int32))
    return partials.sum()
